# Initial kernel scaffold; baseline (speedup 1.0000x reference)
import jax
import jax.numpy as jnp
from jax.experimental import pallas as pl


def kernel(x):
    raise NotImplementedError("write your pallas kernel here")


# speedup vs baseline: 27.1582x; 27.1582x over previous
import jax
import jax.numpy as jnp
from jax.experimental import pallas as pl
from jax.experimental.pallas import tpu as pltpu

B = 8
N = 16384
C = 3
NPTS = 1024
BIG = 1e10
NEG = -1e30


def _fps_body(x_ref, out_ref):
    X = x_ref[0]
    Y = x_ref[1]
    Z = x_ref[2]

    iota_n = jax.lax.broadcasted_iota(jnp.int32, (B, N), 1)
    iota_p = jax.lax.broadcasted_iota(jnp.int32, (B, NPTS), 1)

    qx0 = X[:, 0:1]
    qy0 = Y[:, 0:1]
    qz0 = Z[:, 0:1]
    zeros_p = jnp.zeros((B, NPTS), dtype=jnp.float32)
    ox0 = jnp.where(iota_p == 0, qx0, zeros_p)
    oy0 = jnp.where(iota_p == 0, qy0, zeros_p)
    oz0 = jnp.where(iota_p == 0, qz0, zeros_p)
    dists0 = jnp.full((B, N), BIG, dtype=jnp.float32)

    def body(i, carry):
        dists, ox, oy, oz, qx, qy, qz = carry
        dx = X - qx
        dy = Y - qy
        dz = Z - qz
        d = dx * dx + dy * dy + dz * dz
        dists = jnp.minimum(dists, d)
        m = jnp.max(dists, axis=1, keepdims=True)
        nxt = jnp.min(jnp.where(dists == m, iota_n, N), axis=1, keepdims=True)
        sel = iota_n == nxt
        qx = jnp.max(jnp.where(sel, X, NEG), axis=1, keepdims=True)
        qy = jnp.max(jnp.where(sel, Y, NEG), axis=1, keepdims=True)
        qz = jnp.max(jnp.where(sel, Z, NEG), axis=1, keepdims=True)
        osel = iota_p == (i + 1)
        ox = jnp.where(osel, qx, ox)
        oy = jnp.where(osel, qy, oy)
        oz = jnp.where(osel, qz, oz)
        return dists, ox, oy, oz, qx, qy, qz

    carry = (dists0, ox0, oy0, oz0, qx0, qy0, qz0)
    _, ox, oy, oz, _, _, _ = jax.lax.fori_loop(0, NPTS - 1, body, carry)
    out_ref[0] = ox
    out_ref[1] = oy
    out_ref[2] = oz


def kernel(x):
    xt = jnp.transpose(x, (1, 0, 2))
    out = pl.pallas_call(
        _fps_body,
        out_shape=jax.ShapeDtypeStruct((C, B, NPTS), jnp.float32),
    )(xt)
    return jnp.transpose(out, (1, 0, 2))
